# baseline (speedup 1.0000x reference)
import dataclasses
import functools

import jax
import jax.numpy as jnp
from jax import lax
from jax.experimental import pallas as pl
from jax.experimental.pallas import tpu as pltpu
from jax.experimental.pallas import tpu_sc as plsc

N_NODES = 10000
D_FEAT = 128
N_EDGES = 320000

NC = 2
NS = 16
NW = NC * NS
E_TILE = N_EDGES // NW
WIN = 50
NWIN = E_TILE // WIN
ROWS_SUB = 624
ROWS_TAIL = N_NODES - NS * ROWS_SUB
LANES = 16
NRING = 5
KAHEAD = 3
NIDX = 10


def _spmm_partials(src, packed, zeros):
    mesh = plsc.VectorSubcoreMesh(core_axis_name="c", subcore_axis_name="s")
    cp = pltpu.CompilerParams()
    if "needs_layout_passes" in pltpu.CompilerParams.__dataclass_fields__:
        cp = dataclasses.replace(cp, needs_layout_passes=False)

    @functools.partial(
        pl.kernel,
        compiler_params=cp,
        out_type=jax.ShapeDtypeStruct((NC, N_NODES, D_FEAT), jnp.float32),
        mesh=mesh,
        scratch_types=[
            pltpu.VMEM_SHARED((N_NODES, D_FEAT), jnp.float32),
            pltpu.SemaphoreType.DMA,
        ] + [pltpu.VMEM((3, WIN), jnp.int32)] * NIDX
          + [pltpu.VMEM((WIN, D_FEAT), jnp.float32)] * NRING
          + [pltpu.SemaphoreType.DMA] * (NIDX + 2 * NRING),
    )
    def k(src_hbm, pk_hbm, zeros_hbm, out_hbm, acc, sem, *rest):
        pkw = rest[0:NIDX]
        gring = rest[NIDX:NIDX + NRING]
        isem = rest[NIDX + NRING:2 * NIDX + NRING]
        gsem = rest[2 * NIDX + NRING:2 * NIDX + 2 * NRING]
        ssem = rest[2 * NIDX + 2 * NRING:]
        c = lax.axis_index("c")
        s = lax.axis_index("s")
        wid = c * NS + s
        wbase = wid * NWIN

        pltpu.sync_copy(zeros_hbm.at[pl.ds(s * ROWS_SUB, ROWS_SUB)],
                        acc.at[pl.ds(s * ROWS_SUB, ROWS_SUB)])

        @pl.when(s == 0)
        def _():
            pltpu.sync_copy(zeros_hbm.at[pl.ds(NS * ROWS_SUB, ROWS_TAIL)],
                            acc.at[pl.ds(NS * ROWS_SUB, ROWS_TAIL)])

        plsc.subcore_barrier()

        def issue_idx(w, i):
            pltpu.async_copy(pk_hbm.at[wbase + w], pkw[i], isem[i])

        def wait_idx(w, i):
            pltpu.make_async_copy(pk_hbm.at[wbase + w], pkw[i],
                                  isem[i]).wait()

        def issue_gather(w, i, b):
            pltpu.async_copy(src_hbm.at[pkw[i].at[0]], gring[b], gsem[b])

        def wait_gather(w, i, b):
            pltpu.make_async_copy(src_hbm.at[pkw[i].at[0]], gring[b],
                                  gsem[b]).wait()

        def issue_scatter(w, i, b):
            pltpu.async_copy(gring[b], acc.at[pkw[i].at[1]], ssem[b],
                             add=True)

        def wait_scatter(w, i, b):
            pltpu.make_async_copy(gring[b], acc.at[pkw[i].at[1]],
                                  ssem[b]).wait()

        for w in range(NIDX):
            issue_idx(w, w)
        for w in range(KAHEAD):
            wait_idx(w, w)
            issue_gather(w, w, w)

        @pl.loop(0, NWIN, step=NIDX)
        def _(w0):
            for i in range(NIDX):
                w = w0 + i
                b = i % NRING
                wait_gather(w, i, b)

                gbuf = gring[b]
                pki = pkw[i]

                @plsc.parallel_loop(0, WIN // 10 - 1, unroll=2)
                def _(g):
                    e0 = g * 10
                    vv = plsc.bitcast(pki[2, pl.ds(e0, LANES)], jnp.float32)
                    for u in range(10):
                        vbc = jnp.broadcast_to(vv[u], (LANES,))
                        for j in range(D_FEAT // LANES):
                            sl = (e0 + u, pl.ds(j * LANES, LANES))
                            gbuf[sl] = gbuf[sl] * vbc

                vvf = plsc.bitcast(pki[2, pl.ds(WIN - LANES, LANES)],
                                   jnp.float32)
                for u in range(WIN - LANES + 6, WIN - LANES + LANES):
                    vbc = jnp.broadcast_to(vvf[u - (WIN - LANES)], (LANES,))
                    for j in range(D_FEAT // LANES):
                        sl = (u, pl.ds(j * LANES, LANES))
                        gbuf[sl] = gbuf[sl] * vbc

                issue_scatter(w, i, b)

                i2 = (i - 2) % NIDX
                b2 = (i - 2) % NRING

                @pl.when(w >= 2)
                def _():
                    wait_scatter(w - 2, i2, b2)

                @pl.when(jnp.logical_and(w >= 2, w + NIDX - 2 < NWIN))
                def _():
                    issue_idx(w + NIDX - 2, i2)

                i3 = (i + KAHEAD) % NIDX

                @pl.when(w + KAHEAD < NWIN)
                def _():
                    wait_idx(w + KAHEAD, i3)
                    issue_gather(w + KAHEAD, i3, b2)

        wait_scatter(NWIN - 2, (NWIN - 2) % NIDX, (NWIN - 2) % NRING)
        wait_scatter(NWIN - 1, (NWIN - 1) % NIDX, (NWIN - 1) % NRING)

        plsc.subcore_barrier()
        pltpu.sync_copy(acc.at[pl.ds(s * ROWS_SUB, ROWS_SUB)],
                        out_hbm.at[c].at[pl.ds(s * ROWS_SUB, ROWS_SUB)])

        @pl.when(s == 0)
        def _():
            pltpu.sync_copy(acc.at[pl.ds(NS * ROWS_SUB, ROWS_TAIL)],
                            out_hbm.at[c].at[pl.ds(NS * ROWS_SUB, ROWS_TAIL)])

    return k(src, packed, zeros)


def _combine_scaled(p0, p1, alph):
    def body(a_ref, p0_ref, p1_ref, o_ref):
        o_ref[...] = a_ref[0, 0] * (p0_ref[...] + p1_ref[...])

    return pl.pallas_call(
        body,
        out_shape=jax.ShapeDtypeStruct((N_NODES, D_FEAT), jnp.float32),
        in_specs=[
            pl.BlockSpec(memory_space=pltpu.SMEM),
            pl.BlockSpec(),
            pl.BlockSpec(),
        ],
        out_specs=pl.BlockSpec(),
    )(alph, p0, p1)


def _finalize(q0, q1, x):
    def body(q0_ref, q1_ref, x_ref, o_ref):
        xc = jnp.nan_to_num(x_ref[...], nan=0.0, posinf=1e6, neginf=-1e6)
        o_ref[...] = jnp.clip((q0_ref[...] + q1_ref[...]) - xc, -5.0, 5.0)

    return pl.pallas_call(
        body,
        out_shape=jax.ShapeDtypeStruct((N_NODES, D_FEAT), jnp.float32),
    )(q0, q1, x)


def kernel(t, x, rows, cols, vals, alpha_train, temperature):
    del t
    cols2d = cols.astype(jnp.int32).reshape(N_EDGES // WIN, WIN)
    rows2d = rows.astype(jnp.int32).reshape(N_EDGES // WIN, WIN)
    vals_i = jax.lax.bitcast_convert_type(
        vals.astype(jnp.float32).reshape(N_EDGES // WIN, WIN), jnp.int32)
    packed = jnp.stack([cols2d, rows2d, vals_i], axis=1)
    zeros = jnp.zeros((N_NODES, D_FEAT), jnp.float32)
    alph = jax.nn.sigmoid(alpha_train * temperature).reshape(1, 1)

    p = _spmm_partials(x, packed, zeros)
    ax = _combine_scaled(p[0], p[1], alph)
    q = _spmm_partials(ax, packed, zeros)
    return _finalize(q[0], q[1], x)


# speedup vs baseline: 1.0267x; 1.0267x over previous
import dataclasses
import functools

import jax
import jax.numpy as jnp
from jax import lax
from jax.experimental import pallas as pl
from jax.experimental.pallas import tpu as pltpu
from jax.experimental.pallas import tpu_sc as plsc

N_NODES = 10000
D_FEAT = 128
N_EDGES = 320000

NC = 2
NS = 16
NW = NC * NS
E_TILE = N_EDGES // NW
WIN = 50
NWIN = E_TILE // WIN
ROWS_SUB = 624
ROWS_TAIL = N_NODES - NS * ROWS_SUB
LANES = 16
NRING = 5
KAHEAD = 4
NIDX = 10


def _spmm_partials(src, packed, zeros):
    mesh = plsc.VectorSubcoreMesh(core_axis_name="c", subcore_axis_name="s")
    cp = pltpu.CompilerParams()
    if "needs_layout_passes" in pltpu.CompilerParams.__dataclass_fields__:
        cp = dataclasses.replace(cp, needs_layout_passes=False)

    @functools.partial(
        pl.kernel,
        compiler_params=cp,
        out_type=jax.ShapeDtypeStruct((NC, N_NODES, D_FEAT), jnp.float32),
        mesh=mesh,
        scratch_types=[
            pltpu.VMEM_SHARED((N_NODES, D_FEAT), jnp.float32),
            pltpu.SemaphoreType.DMA,
        ] + [pltpu.VMEM((3, WIN), jnp.int32)] * NIDX
          + [pltpu.VMEM((WIN, D_FEAT), jnp.float32)] * NRING
          + [pltpu.SemaphoreType.DMA] * (NIDX + 2 * NRING),
    )
    def k(src_hbm, pk_hbm, zeros_hbm, out_hbm, acc, sem, *rest):
        pkw = rest[0:NIDX]
        gring = rest[NIDX:NIDX + NRING]
        isem = rest[NIDX + NRING:2 * NIDX + NRING]
        gsem = rest[2 * NIDX + NRING:2 * NIDX + 2 * NRING]
        ssem = rest[2 * NIDX + 2 * NRING:]
        c = lax.axis_index("c")
        s = lax.axis_index("s")
        wid = c * NS + s
        wbase = wid * NWIN

        pltpu.sync_copy(zeros_hbm.at[pl.ds(s * ROWS_SUB, ROWS_SUB)],
                        acc.at[pl.ds(s * ROWS_SUB, ROWS_SUB)])

        @pl.when(s == 0)
        def _():
            pltpu.sync_copy(zeros_hbm.at[pl.ds(NS * ROWS_SUB, ROWS_TAIL)],
                            acc.at[pl.ds(NS * ROWS_SUB, ROWS_TAIL)])

        plsc.subcore_barrier()

        def issue_idx(w, i):
            pltpu.async_copy(pk_hbm.at[wbase + w], pkw[i], isem[i])

        def wait_idx(w, i):
            pltpu.make_async_copy(pk_hbm.at[wbase + w], pkw[i],
                                  isem[i]).wait()

        def issue_gather(w, i, b):
            pltpu.async_copy(src_hbm.at[pkw[i].at[0]], gring[b], gsem[b])

        def wait_gather(w, i, b):
            pltpu.make_async_copy(src_hbm.at[pkw[i].at[0]], gring[b],
                                  gsem[b]).wait()

        def issue_scatter(w, i, b):
            pltpu.async_copy(gring[b], acc.at[pkw[i].at[1]], ssem[b],
                             add=True)

        def wait_scatter(w, i, b):
            pltpu.make_async_copy(gring[b], acc.at[pkw[i].at[1]],
                                  ssem[b]).wait()

        for w in range(NIDX):
            issue_idx(w, w)
        for w in range(KAHEAD):
            wait_idx(w, w)
            issue_gather(w, w, w)

        @pl.loop(0, NWIN, step=NIDX)
        def _(w0):
            for i in range(NIDX):
                w = w0 + i
                b = i % NRING
                wait_gather(w, i, b)

                gbuf = gring[b]
                pki = pkw[i]

                @plsc.parallel_loop(0, WIN // 10 - 1, unroll=2)
                def _(g):
                    e0 = g * 10
                    vv = plsc.bitcast(pki[2, pl.ds(e0, LANES)], jnp.float32)
                    for u in range(10):
                        vbc = jnp.broadcast_to(vv[u], (LANES,))
                        for j in range(D_FEAT // LANES):
                            sl = (e0 + u, pl.ds(j * LANES, LANES))
                            gbuf[sl] = gbuf[sl] * vbc

                vvf = plsc.bitcast(pki[2, pl.ds(WIN - LANES, LANES)],
                                   jnp.float32)
                for u in range(WIN - LANES + 6, WIN - LANES + LANES):
                    vbc = jnp.broadcast_to(vvf[u - (WIN - LANES)], (LANES,))
                    for j in range(D_FEAT // LANES):
                        sl = (u, pl.ds(j * LANES, LANES))
                        gbuf[sl] = gbuf[sl] * vbc

                issue_scatter(w, i, b)

                i2 = (i - 1) % NIDX
                b2 = (i - 1) % NRING

                @pl.when(w >= 1)
                def _():
                    wait_scatter(w - 1, i2, b2)

                @pl.when(jnp.logical_and(w >= 1, w + NIDX - 1 < NWIN))
                def _():
                    issue_idx(w + NIDX - 1, i2)

                i3 = (i + KAHEAD) % NIDX

                @pl.when(w + KAHEAD < NWIN)
                def _():
                    wait_idx(w + KAHEAD, i3)
                    issue_gather(w + KAHEAD, i3, b2)

        wait_scatter(NWIN - 1, (NWIN - 1) % NIDX, (NWIN - 1) % NRING)

        plsc.subcore_barrier()
        pltpu.sync_copy(acc.at[pl.ds(s * ROWS_SUB, ROWS_SUB)],
                        out_hbm.at[c].at[pl.ds(s * ROWS_SUB, ROWS_SUB)])

        @pl.when(s == 0)
        def _():
            pltpu.sync_copy(acc.at[pl.ds(NS * ROWS_SUB, ROWS_TAIL)],
                            out_hbm.at[c].at[pl.ds(NS * ROWS_SUB, ROWS_TAIL)])

    return k(src, packed, zeros)


def _combine_scaled(p0, p1, alph):
    def body(a_ref, p0_ref, p1_ref, o_ref):
        o_ref[...] = a_ref[0, 0] * (p0_ref[...] + p1_ref[...])

    return pl.pallas_call(
        body,
        out_shape=jax.ShapeDtypeStruct((N_NODES, D_FEAT), jnp.float32),
        in_specs=[
            pl.BlockSpec(memory_space=pltpu.SMEM),
            pl.BlockSpec(),
            pl.BlockSpec(),
        ],
        out_specs=pl.BlockSpec(),
    )(alph, p0, p1)


def _finalize(q0, q1, x):
    def body(q0_ref, q1_ref, x_ref, o_ref):
        xc = jnp.nan_to_num(x_ref[...], nan=0.0, posinf=1e6, neginf=-1e6)
        o_ref[...] = jnp.clip((q0_ref[...] + q1_ref[...]) - xc, -5.0, 5.0)

    return pl.pallas_call(
        body,
        out_shape=jax.ShapeDtypeStruct((N_NODES, D_FEAT), jnp.float32),
    )(q0, q1, x)


def kernel(t, x, rows, cols, vals, alpha_train, temperature):
    del t
    cols2d = cols.astype(jnp.int32).reshape(N_EDGES // WIN, WIN)
    rows2d = rows.astype(jnp.int32).reshape(N_EDGES // WIN, WIN)
    vals_i = jax.lax.bitcast_convert_type(
        vals.astype(jnp.float32).reshape(N_EDGES // WIN, WIN), jnp.int32)
    packed = jnp.stack([cols2d, rows2d, vals_i], axis=1)
    zeros = jnp.zeros((N_NODES, D_FEAT), jnp.float32)
    alph = jax.nn.sigmoid(alpha_train * temperature).reshape(1, 1)

    p = _spmm_partials(x, packed, zeros)
    ax = _combine_scaled(p[0], p[1], alph)
    q = _spmm_partials(ax, packed, zeros)
    return _finalize(q[0], q[1], x)


# speedup vs baseline: 1.0641x; 1.0364x over previous
import dataclasses
import functools

import jax
import jax.numpy as jnp
from jax import lax
from jax.experimental import pallas as pl
from jax.experimental.pallas import tpu as pltpu
from jax.experimental.pallas import tpu_sc as plsc

N_NODES = 10000
D_FEAT = 128
N_EDGES = 320000

NC = 2
NS = 16
NW = NC * NS
E_TILE = N_EDGES // NW
WIN = 50
NWIN = E_TILE // WIN
ROWS_SUB = 624
ROWS_TAIL = N_NODES - NS * ROWS_SUB
LANES = 16
NRING = 5
KAHEAD = 4
NIDX = 10


def _spmm_partials(src, packed, zeros):
    mesh = plsc.VectorSubcoreMesh(core_axis_name="c", subcore_axis_name="s")
    cp = pltpu.CompilerParams()
    if "needs_layout_passes" in pltpu.CompilerParams.__dataclass_fields__:
        cp = dataclasses.replace(cp, needs_layout_passes=False)

    @functools.partial(
        pl.kernel,
        compiler_params=cp,
        out_type=jax.ShapeDtypeStruct((NC, N_NODES, D_FEAT), jnp.float32),
        mesh=mesh,
        scratch_types=[
            pltpu.VMEM_SHARED((N_NODES, D_FEAT), jnp.float32),
            pltpu.SemaphoreType.DMA,
        ] + [pltpu.VMEM((3, WIN), jnp.int32)] * NIDX
          + [pltpu.VMEM((WIN, D_FEAT), jnp.float32)] * NRING
          + [pltpu.SemaphoreType.DMA] * (NIDX + 2 * NRING),
    )
    def k(src_hbm, pk_hbm, zeros_hbm, out_hbm, acc, sem, *rest):
        pkw = rest[0:NIDX]
        gring = rest[NIDX:NIDX + NRING]
        isem = rest[NIDX + NRING:2 * NIDX + NRING]
        gsem = rest[2 * NIDX + NRING:2 * NIDX + 2 * NRING]
        ssem = rest[2 * NIDX + 2 * NRING:]
        c = lax.axis_index("c")
        s = lax.axis_index("s")
        wid = c * NS + s
        wbase = wid * NWIN

        def issue_idx(w, i):
            pltpu.async_copy(pk_hbm.at[wbase + w], pkw[i], isem[i])

        def wait_idx(w, i):
            pltpu.make_async_copy(pk_hbm.at[wbase + w], pkw[i],
                                  isem[i]).wait()

        def issue_gather(w, i, b):
            pltpu.async_copy(src_hbm.at[pkw[i].at[0]], gring[b], gsem[b])

        def wait_gather(w, i, b):
            pltpu.make_async_copy(src_hbm.at[pkw[i].at[0]], gring[b],
                                  gsem[b]).wait()

        def issue_scatter(w, i, b):
            pltpu.async_copy(gring[b], acc.at[pkw[i].at[1]], ssem[b],
                             add=True)

        def wait_scatter(w, i, b):
            pltpu.make_async_copy(gring[b], acc.at[pkw[i].at[1]],
                                  ssem[b]).wait()

        for w in range(NIDX):
            issue_idx(w, w)
        for w in range(KAHEAD):
            wait_idx(w, w)
            issue_gather(w, w, w)

        pltpu.sync_copy(zeros_hbm.at[pl.ds(s * ROWS_SUB, ROWS_SUB)],
                        acc.at[pl.ds(s * ROWS_SUB, ROWS_SUB)])

        @pl.when(s == 0)
        def _():
            pltpu.sync_copy(zeros_hbm.at[pl.ds(NS * ROWS_SUB, ROWS_TAIL)],
                            acc.at[pl.ds(NS * ROWS_SUB, ROWS_TAIL)])

        plsc.subcore_barrier()

        @pl.loop(0, NWIN, step=NIDX)
        def _(w0):
            for i in range(NIDX):
                w = w0 + i
                b = i % NRING
                wait_gather(w, i, b)

                gbuf = gring[b]
                pki = pkw[i]

                @plsc.parallel_loop(0, WIN // 10 - 1, unroll=2)
                def _(g):
                    e0 = g * 10
                    vv = plsc.bitcast(pki[2, pl.ds(e0, LANES)], jnp.float32)
                    for u in range(10):
                        vbc = jnp.broadcast_to(vv[u], (LANES,))
                        for j in range(D_FEAT // LANES):
                            sl = (e0 + u, pl.ds(j * LANES, LANES))
                            gbuf[sl] = gbuf[sl] * vbc

                vvf = plsc.bitcast(pki[2, pl.ds(WIN - LANES, LANES)],
                                   jnp.float32)
                for u in range(WIN - LANES + 6, WIN - LANES + LANES):
                    vbc = jnp.broadcast_to(vvf[u - (WIN - LANES)], (LANES,))
                    for j in range(D_FEAT // LANES):
                        sl = (u, pl.ds(j * LANES, LANES))
                        gbuf[sl] = gbuf[sl] * vbc

                issue_scatter(w, i, b)

                i2 = (i - 1) % NIDX
                b2 = (i - 1) % NRING

                @pl.when(w >= 1)
                def _():
                    wait_scatter(w - 1, i2, b2)

                @pl.when(jnp.logical_and(w >= 1, w + NIDX - 1 < NWIN))
                def _():
                    issue_idx(w + NIDX - 1, i2)

                i3 = (i + KAHEAD) % NIDX

                @pl.when(w + KAHEAD < NWIN)
                def _():
                    wait_idx(w + KAHEAD, i3)
                    issue_gather(w + KAHEAD, i3, b2)

        wait_scatter(NWIN - 1, (NWIN - 1) % NIDX, (NWIN - 1) % NRING)

        plsc.subcore_barrier()
        pltpu.sync_copy(acc.at[pl.ds(s * ROWS_SUB, ROWS_SUB)],
                        out_hbm.at[c].at[pl.ds(s * ROWS_SUB, ROWS_SUB)])

        @pl.when(s == 0)
        def _():
            pltpu.sync_copy(acc.at[pl.ds(NS * ROWS_SUB, ROWS_TAIL)],
                            out_hbm.at[c].at[pl.ds(NS * ROWS_SUB, ROWS_TAIL)])

    return k(src, packed, zeros)


def _combine_scaled(p0, p1, alph):
    def body(a_ref, p0_ref, p1_ref, o_ref):
        o_ref[...] = a_ref[0, 0] * (p0_ref[...] + p1_ref[...])

    return pl.pallas_call(
        body,
        out_shape=jax.ShapeDtypeStruct((N_NODES, D_FEAT), jnp.float32),
        in_specs=[
            pl.BlockSpec(memory_space=pltpu.SMEM),
            pl.BlockSpec(),
            pl.BlockSpec(),
        ],
        out_specs=pl.BlockSpec(),
    )(alph, p0, p1)


def _finalize(q0, q1, x):
    def body(q0_ref, q1_ref, x_ref, o_ref):
        xc = jnp.nan_to_num(x_ref[...], nan=0.0, posinf=1e6, neginf=-1e6)
        o_ref[...] = jnp.clip((q0_ref[...] + q1_ref[...]) - xc, -5.0, 5.0)

    return pl.pallas_call(
        body,
        out_shape=jax.ShapeDtypeStruct((N_NODES, D_FEAT), jnp.float32),
    )(q0, q1, x)


def kernel(t, x, rows, cols, vals, alpha_train, temperature):
    del t
    cols2d = cols.astype(jnp.int32).reshape(N_EDGES // WIN, WIN)
    rows2d = rows.astype(jnp.int32).reshape(N_EDGES // WIN, WIN)
    vals_i = jax.lax.bitcast_convert_type(
        vals.astype(jnp.float32).reshape(N_EDGES // WIN, WIN), jnp.int32)
    packed = jnp.stack([cols2d, rows2d, vals_i], axis=1)
    zeros = jnp.zeros((N_NODES, D_FEAT), jnp.float32)
    alph = jax.nn.sigmoid(alpha_train * temperature).reshape(1, 1)

    p = _spmm_partials(x, packed, zeros)
    ax = _combine_scaled(p[0], p[1], alph)
    q = _spmm_partials(ax, packed, zeros)
    return _finalize(q[0], q[1], x)
